# Initial kernel scaffold; baseline (speedup 1.0000x reference)
import jax
import jax.numpy as jnp
from jax.experimental import pallas as pl


def kernel(x, table):
    raise NotImplementedError("write your pallas kernel here")


# speedup vs baseline: 4.9795x; 4.9795x over previous
import functools

import jax
import jax.numpy as jnp
from jax import lax
from jax.experimental import pallas as pl
from jax.experimental.pallas import tpu as pltpu
from jax.experimental.pallas import tpu_sc as plsc

OUT_DIM = 64

NC = 2
NS = 16
NW = NC * NS

K = 8
CHUNK = K * 128


@functools.lru_cache(maxsize=None)
def _gather_kernel(B):
    rows_per_w = B // NW
    n_outer = rows_per_w // CHUNK
    mesh = plsc.VectorSubcoreMesh(core_axis_name="c", subcore_axis_name="s")

    @functools.partial(
        pl.kernel,
        mesh=mesh,
        out_type=jax.ShapeDtypeStruct((B, OUT_DIM), jnp.float32),
        scratch_types=[
            pltpu.VMEM((K, 128), jnp.int32),
            pltpu.VMEM((CHUNK, OUT_DIM), jnp.float32),
            pltpu.SemaphoreType.DMA,
        ],
        compiler_params=pltpu.CompilerParams(use_tc_tiling_on_sc=False),
    )
    def k(idx_hbm, table_hbm, out_hbm, idx_v, rows_v, sem):
        wid = lax.axis_index("s") * NC + lax.axis_index("c")
        iblk0 = wid * (rows_per_w // 128)
        row0 = wid * rows_per_w

        def body(i, carry):
            pltpu.sync_copy(idx_hbm.at[pl.ds(iblk0 + i * K, K)], idx_v)
            cps = [
                pltpu.async_copy(
                    table_hbm.at[idx_v.at[j]],
                    rows_v.at[pl.ds(j * 128, 128)],
                    sem,
                )
                for j in range(K)
            ]
            for c in cps:
                c.wait()
            pltpu.sync_copy(rows_v, out_hbm.at[pl.ds(row0 + i * CHUNK, CHUNK)])
            return carry

        lax.fori_loop(0, n_outer, body, 0)

    return k


def kernel(x, table):
    orig_shape = x.shape
    B = x.size
    idx = x.reshape(B // 128, 128).astype(jnp.int32)
    out = _gather_kernel(B)(idx, table)
    return out.reshape(*orig_shape, OUT_DIM)
